# baseline (speedup 1.0000x reference)
import functools

import jax
import jax.numpy as jnp
import numpy as np
from jax import lax
from jax.experimental import pallas as pl
from jax.experimental.pallas import tpu as pltpu
from jax.experimental.pallas import tpu_sc as plsc

N = 10000
K = 8
E = 160000
G = 64
NHID = 64
NL_RHO = 2
NHEAD = 4
DFF = 128
DH = NHID // NHEAD

NC = 2
NS = 16

_SC_MESH = plsc.VectorSubcoreMesh(core_axis_name="c", subcore_axis_name="s")

_CH = 624
_CH_TAIL = N - NS * _CH

_B = 128
_EPW0 = (E // (NC * NS * _B)) * _B
_NB0 = _EPW0 // _B
_TAIL0 = (E - NC * NS * _EPW0) // _B


def _rowchunk_copy(src_ref, dst_ref, sub):
    pltpu.sync_copy(src_ref.at[pl.ds(sub * _CH, _CH)],
                    dst_ref.at[pl.ds(sub * _CH, _CH)])

    @pl.when(sub == 0)
    def _():
        pltpu.sync_copy(src_ref.at[pl.ds(NS * _CH, _CH_TAIL)],
                        dst_ref.at[pl.ds(NS * _CH, _CH_TAIL)])


def _edge_batch(src_ref, dst_ref, table_ref, sidx, didx, gbuf, acc, gsem, base):
    pltpu.sync_copy(src_ref.at[pl.ds(base, _B)], sidx)
    pltpu.sync_copy(dst_ref.at[pl.ds(base, _B)], didx)
    pltpu.async_copy(table_ref.at[sidx], gbuf, gsem).wait()
    pltpu.sync_copy(gbuf, acc.at[didx], add=True)


def _agg0_body(ev_hbm, src_hbm, dst_hbm, zero_hbm, out_hbm,
               sidx, didx, gbuf, acc, gsem):
    core = lax.axis_index("c")
    sub = lax.axis_index("s")
    wid = sub * NC + core
    _rowchunk_copy(zero_hbm, acc, sub)
    plsc.subcore_barrier()

    def body(i, _):
        _edge_batch(src_hbm, dst_hbm, ev_hbm, sidx, didx, gbuf, acc, gsem,
                    wid * _EPW0 + i * _B)
        return 0

    lax.fori_loop(0, _NB0, body, 0)

    @pl.when(wid < _TAIL0)
    def _():
        _edge_batch(src_hbm, dst_hbm, ev_hbm, sidx, didx, gbuf, acc, gsem,
                    NC * NS * _EPW0 + wid * _B)

    plsc.subcore_barrier()
    _rowchunk_copy(acc, out_hbm.at[core], sub)


def _sc_agg0(ev, src, dst, zeros_nk):
    kfn = pl.kernel(
        _agg0_body,
        mesh=_SC_MESH,
        compiler_params=pltpu.CompilerParams(use_tc_tiling_on_sc=False),
        out_type=jax.ShapeDtypeStruct((NC, N, K), jnp.float32),
        scratch_types=[
            pltpu.VMEM((_B,), jnp.int32),
            pltpu.VMEM((_B,), jnp.int32),
            pltpu.VMEM((_B, K), jnp.float32),
            pltpu.VMEM_SHARED((N, K), jnp.float32),
            pltpu.SemaphoreType.DMA,
        ],
    )
    return kfn(ev, src, dst, zeros_nk)


_EP = NS * 80 * _B
_NB1 = _EP // (NS * _B)
_LAST_VALID = (E - (NS - 1) * _NB1 * _B) // _B


def _agg1_body(x_hbm, srck_hbm, dst_hbm, zero_hbm, out_hbm,
               sidx_all, didx0, didx1, gbuf0, gbuf1, acc, sem0, sem1):
    core = lax.axis_index("c")
    sub = lax.axis_index("s")

    for kk in range(K // NC):
        k = core * (K // NC) + kk
        _rowchunk_copy(zero_hbm, acc, sub)
        pltpu.sync_copy(srck_hbm.at[k, pl.ds(sub * _NB1, _NB1)], sidx_all)
        plsc.subcore_barrier()

        def start(i, g, sem):
            return pltpu.async_copy(x_hbm.at[sidx_all.at[i]], g, sem)

        start(0, gbuf0, sem0)

        nvalid = E // (NS * _B) - sub * _NB1 // NS

        def body(i2, _):
            a = 2 * i2
            start(a + 1, gbuf1, sem1)
            pltpu.sync_copy(dst_hbm.at[sub * _NB1 + a, 0], didx0)
            pltpu.make_async_copy(x_hbm.at[sidx_all.at[a]], gbuf0, sem0).wait()

            @pl.when((sub < NS - 1) | (a < _LAST_VALID))
            def _():
                pltpu.sync_copy(gbuf0, acc.at[didx0], add=True)

            @pl.when(i2 < _NB1 // 2 - 1)
            def _():
                start(a + 2, gbuf0, sem0)

            pltpu.sync_copy(dst_hbm.at[sub * _NB1 + a + 1, 0], didx1)
            pltpu.make_async_copy(x_hbm.at[sidx_all.at[a + 1]], gbuf1,
                                  sem1).wait()

            @pl.when((sub < NS - 1) | (a + 1 < _LAST_VALID))
            def _():
                pltpu.sync_copy(gbuf1, acc.at[didx1], add=True)
            return 0

        lax.fori_loop(0, _NB1 // 2, body, 0)
        plsc.subcore_barrier()
        _rowchunk_copy(acc, out_hbm.at[k], sub)
        plsc.subcore_barrier()


def _sc_agg1(x1cat_flat, srck, dst, zeros_big):
    srck = srck.reshape(K, _EP // _B, _B)
    dst = dst.reshape(_EP // _B, 1, _B)
    kfn = pl.kernel(
        _agg1_body,
        mesh=_SC_MESH,
        compiler_params=pltpu.CompilerParams(use_tc_tiling_on_sc=False),
        out_type=jax.ShapeDtypeStruct((K, N, 2 * NHID), jnp.float32),
        scratch_types=[
            pltpu.VMEM((_NB1, _B), jnp.int32),
            pltpu.VMEM((_B,), jnp.int32),
            pltpu.VMEM((_B,), jnp.int32),
            pltpu.VMEM((_B, 2 * NHID), jnp.float32),
            pltpu.VMEM((_B, 2 * NHID), jnp.float32),
            pltpu.VMEM_SHARED((N, 2 * NHID), jnp.float32),
            pltpu.SemaphoreType.DMA,
            pltpu.SemaphoreType.DMA,
        ],
    )
    return kfn(x1cat_flat, srck, dst, zeros_big)


def _dot(a, b):
    return jnp.dot(a, b, precision=jax.lax.Precision.DEFAULT)


def _dot_hi(a, b):
    return jnp.dot(a, b, precision=jax.lax.Precision.DEFAULT)


def _b16(x):
    return x.astype(jnp.bfloat16).astype(jnp.float32)

_BN1 = 2000
_BN = 400


def _t1_kernel(ev_ref, parts_ref, maskf_ref, s_ref, stats_ref):
    i = pl.program_id(0)
    s = ev_ref[...] + parts_ref[0] + parts_ref[1]
    mk = maskf_ref[...]
    sm = s * mk

    @pl.when(i == 0)
    def _():
        stats_ref[...] = jnp.zeros_like(stats_ref)

    stats_ref[0, :] += jnp.sum(sm, axis=0)
    stats_ref[1, :] += jnp.sum(sm * s, axis=0)
    stats_ref[2, :] += jnp.sum(mk, axis=0)
    s_ref[...] = s


def _t1_stage(ev, parts, maskf):
    grid = N // _BN1
    return pl.pallas_call(
        _t1_kernel,
        grid=(grid,),
        in_specs=[
            pl.BlockSpec((_BN1, K), lambda i: (i, 0)),
            pl.BlockSpec((NC, _BN1, K), lambda i: (0, i, 0)),
            pl.BlockSpec((_BN1, K), lambda i: (i, 0)),
        ],
        out_specs=[
            pl.BlockSpec((_BN1, K), lambda i: (i, 0)),
            pl.BlockSpec((3, K), lambda i: (0, 0)),
        ],
        out_shape=[
            jax.ShapeDtypeStruct((N, K), jnp.float32),
            jax.ShapeDtypeStruct((3, K), jnp.float32),
        ],
    )(ev, parts, maskf)


def _t2_kernel(sT_ref, maskT_ref, a2_ref, b2_ref, x1_ref):
    s3 = sT_ref[...]
    y = s3 * a2_ref[0, :] + b2_ref[0, :]
    x1_ref[...] = jnp.maximum(y, 0.0) * maskT_ref[...]


def _t2_stage(sT, maskT, a2, b2):
    grid = N // _BN
    return pl.pallas_call(
        _t2_kernel,
        grid=(grid,),
        in_specs=[
            pl.BlockSpec((K, _BN, 1), lambda i: (0, i, 0)),
            pl.BlockSpec((K, _BN, 1), lambda i: (0, i, 0)),
            pl.BlockSpec((1, 2 * NHID), lambda i: (0, 0)),
            pl.BlockSpec((1, 2 * NHID), lambda i: (0, 0)),
        ],
        out_specs=pl.BlockSpec((K, _BN, 2 * NHID), lambda i: (0, i, 0)),
        out_shape=jax.ShapeDtypeStruct((K, N, 2 * NHID), jnp.float32),
    )(sT, maskT, a2, b2)


def _t3_kernel(x1_ref, agg_ref, w2_ref, maskT_ref, hcat_ref, stats_ref):
    i = pl.program_id(0)
    xs = x1_ref[...] + agg_ref[...]
    h = _dot(xs.reshape(K * _BN, 2 * NHID), w2_ref[...]).reshape(K, _BN, 2 * NHID)
    h = h * maskT_ref[...]

    @pl.when(i == 0)
    def _():
        stats_ref[...] = jnp.zeros_like(stats_ref)

    hf = h.reshape(K * _BN, 2 * NHID)
    stats_ref[0, :] += jnp.sum(hf, axis=0)
    stats_ref[1, :] += jnp.sum(hf * hf, axis=0)
    hcat_ref[...] = h


def _t3_stage(x1cat, agg1, w2, maskT):
    grid = N // _BN
    return pl.pallas_call(
        _t3_kernel,
        grid=(grid,),
        in_specs=[
            pl.BlockSpec((K, _BN, 2 * NHID), lambda i: (0, i, 0)),
            pl.BlockSpec((K, _BN, 2 * NHID), lambda i: (0, i, 0)),
            pl.BlockSpec((2 * NHID, 2 * NHID), lambda i: (0, 0)),
            pl.BlockSpec((K, _BN, 1), lambda i: (0, i, 0)),
        ],
        out_specs=[
            pl.BlockSpec((K, _BN, 2 * NHID), lambda i: (0, i, 0)),
            pl.BlockSpec((2, 2 * NHID), lambda i: (0, 0)),
        ],
        out_shape=[
            jax.ShapeDtypeStruct((K, N, 2 * NHID), jnp.float32),
            jax.ShapeDtypeStruct((2, 2 * NHID), jnp.float32),
        ],
    )(x1cat, agg1, w2, maskT)


def _layernorm_k(x, g, b):
    m = jnp.mean(x, axis=-1, keepdims=True)
    d = x - m
    v = jnp.mean(d * d, axis=-1, keepdims=True)
    return d / jnp.sqrt(v + 1e-5) * g + b


def _t4_kernel(hcat_ref, x1_ref, maskT_ref, sc2_ref, sh2_ref,
               isum_ref, hrep_ref, tw_refs, outw_ref, y_ref, stats_ref):
    i = pl.program_id(0)
    mk3 = maskT_ref[...]
    x2 = jnp.maximum(hcat_ref[...] * sc2_ref[0, :] + sh2_ref[0, :], 0.0)
    x2 = x2 * mk3 + x1_ref[...]
    x = x2[..., :NHID] + x2[..., NHID:]

    for l in range(NL_RHO):
        (wq, wk, wv, wo, ln1g, ln1b, ff1, ff1b, ff2, ff2b,
         ln2g, ln2b) = tw_refs[12 * l:12 * (l + 1)]
        xf = x.reshape(K * _BN, NHID)
        q = _dot(xf, wq[...]).reshape(K, _BN, NHID)
        kk = _dot(xf, wk[...]).reshape(K, _BN, NHID)
        v = _dot(xf, wv[...]).reshape(K, _BN, NHID)
        qb = _b16(q)
        kb = _b16(kk)
        sc = []
        for j in range(K):
            prod = (qb * kb[j][None, :, :]).reshape(K * _BN, NHID)
            ph = _b16(prod)
            plo = prod - ph
            sj = (_dot(ph, hrep_ref[...])
                  + _dot(plo, hrep_ref[...])).reshape(K, _BN, NHID)
            mj = mk3[j][None, :, :]
            sc.append(jnp.where(mj > 0.0, sj, -1e9))
        m8 = sc[0]
        for j in range(1, K):
            m8 = jnp.maximum(m8, sc[j])
        ssum = None
        es = []
        for j in range(K):
            ej = jnp.exp(sc[j] - m8)
            es.append(ej)
            ssum = ej if ssum is None else ssum + ej
        vb = _b16(v)
        o = None
        for j in range(K):
            contrib = _b16(es[j] / ssum) * vb[j][None, :, :]
            o = contrib if o is None else o + contrib
        o = _dot(o.reshape(K * _BN, NHID), wo[...]).reshape(K, _BN, NHID)
        x = _layernorm_k(x + o, ln1g[0, :], ln1b[0, :])
        xf = x.reshape(K * _BN, NHID)
        f = jnp.maximum(_dot(xf, ff1[...]) + ff1b[0, :], 0.0)
        f = (_dot(f, ff2[...]) + ff2b[0, :]).reshape(K, _BN, NHID)
        x = _layernorm_k(x + f, ln2g[0, :], ln2b[0, :])
        x = x * mk3

    hsum = x[0]
    for j in range(1, K):
        hsum = hsum + x[j]
    y = _dot(hsum, outw_ref[...])

    @pl.when(i == 0)
    def _():
        stats_ref[...] = jnp.zeros_like(stats_ref)

    stats_ref[0, :] += jnp.sum(y, axis=0)
    stats_ref[1, :] += jnp.sum(y * y, axis=0)
    y_ref[...] = y


def _t4_stage(hcat, x1cat, maskT, sc2, sh2, isum, hrep, tws, outw):
    grid = N // _BN
    full = lambda shape: pl.BlockSpec(shape, lambda i: tuple(0 for _ in shape))
    nblk3 = pl.BlockSpec((K, _BN, 2 * NHID), lambda i: (0, i, 0))

    def body(hcat_ref, x1_ref, maskT_ref, sc2_ref, sh2_ref, isum_ref,
             hrep_ref, *rest):
        tw_refs = rest[:-3]
        outw_ref, y_ref, stats_ref = rest[-3:]
        _t4_kernel(hcat_ref, x1_ref, maskT_ref, sc2_ref, sh2_ref,
                   isum_ref, hrep_ref, tw_refs, outw_ref, y_ref, stats_ref)

    in_specs = [
        nblk3, nblk3,
        pl.BlockSpec((K, _BN, 1), lambda i: (0, i, 0)),
        full((1, 2 * NHID)), full((1, 2 * NHID)),
        full((2 * NHID, NHID)), full((NHID, NHID)),
    ] + [full(t.shape) for t in tws] + [full((NHID, NHID))]
    return pl.pallas_call(
        body,
        grid=(grid,),
        in_specs=in_specs,
        out_specs=[
            pl.BlockSpec((_BN, NHID), lambda i: (i, 0)),
            pl.BlockSpec((2, NHID), lambda i: (0, 0)),
        ],
        out_shape=[
            jax.ShapeDtypeStruct((N, NHID), jnp.float32),
            jax.ShapeDtypeStruct((2, NHID), jnp.float32),
        ],
    )(hcat, x1cat, maskT, sc2, sh2, isum, hrep, *tws, outw)


def _t5_kernel(y_ref, stat_ref, g_ref, b_ref, o_ref):
    s1 = stat_ref[0, :]
    s2 = stat_ref[1, :]
    mean = s1 / N
    var = s2 / N - mean * mean
    o_ref[...] = (y_ref[...] - mean) / jnp.sqrt(var + 1e-5) * g_ref[...] + b_ref[...]


def _t5_stage(y, stats, g, b):
    grid = N // _BN1
    return pl.pallas_call(
        _t5_kernel,
        grid=(grid,),
        in_specs=[
            pl.BlockSpec((_BN1, NHID), lambda i: (i, 0)),
            pl.BlockSpec((2, NHID), lambda i: (0, 0)),
            pl.BlockSpec((NHID,), lambda i: (0,)),
            pl.BlockSpec((NHID,), lambda i: (0,)),
        ],
        out_specs=pl.BlockSpec((_BN1, NHID), lambda i: (i, 0)),
        out_shape=jax.ShapeDtypeStruct((N, NHID), jnp.float32),
    )(y, stats, g, b)


def kernel(eigen_vectors, eigen_values, batch, edge_index, params):
    f32 = jnp.float32
    ev = eigen_vectors.astype(f32)
    src = edge_index[0].astype(jnp.int32)
    dst = edge_index[1].astype(jnp.int32)
    batch = batch.astype(jnp.int32)

    size = (jnp.searchsorted(batch, jnp.arange(1, G + 1, dtype=jnp.int32))
            - jnp.searchsorted(batch, jnp.arange(G, dtype=jnp.int32)))
    cntn = size[batch]
    ar = jnp.arange(K, dtype=jnp.int32)
    maskf = (ar[None, :] < cntn[:, None]).astype(f32)
    maskT3 = (ar[:, None] < cntn[None, :]).astype(f32)[..., None]

    zeros_nk = jnp.zeros((N, K), f32)
    parts = _sc_agg0(ev, src, dst, zeros_nk)
    s, st1 = _t1_stage(ev, parts, maskf)
    s1 = jnp.sum(st1[0]); s2 = jnp.sum(st1[1])
    cnt = jnp.maximum(jnp.sum(st1[2]), 1.0)

    w0 = params['conv_W'][0][0]
    g0 = params['bn_g'][0]; b0 = params['bn_b'][0]
    mu = s1 / cnt
    var0 = w0 * w0 * jnp.maximum(s2 / cnt - mu * mu, 0.0)
    inv0 = g0 / jnp.sqrt(var0 + 1e-5)
    a_p = w0 * inv0
    b_p = b0 - w0 * mu * inv0
    b_m = b0 + w0 * mu * inv0
    a2 = jnp.concatenate([a_p, -a_p])[None, :]
    b2 = jnp.concatenate([b_p, b_m])[None, :]

    x1cat = _t2_stage(s.T[..., None], maskT3, a2, b2)

    pad = _EP - E
    src_p = jnp.concatenate([src, jnp.zeros((pad,), jnp.int32)])
    dst_p = jnp.concatenate([dst, jnp.zeros((pad,), jnp.int32)])
    srck = src_p[None, :] + (jnp.arange(K, dtype=jnp.int32) * N)[:, None]
    zeros_big = jnp.zeros((N, 2 * NHID), f32)
    agg1 = _sc_agg1(x1cat.reshape(K * N, 2 * NHID), srck, dst_p, zeros_big)

    w1 = params['conv_W'][1]
    zz = jnp.zeros_like(w1)
    w2 = jnp.block([[w1, zz], [zz, w1]])
    hcat, st3 = _t3_stage(x1cat, agg1, w2, maskT3)

    g1 = params['bn_g'][1]; b1 = params['bn_b'][1]
    mean3 = st3[0] / cnt
    var3 = jnp.maximum(st3[1] / cnt - mean3 * mean3, 0.0)
    g2 = jnp.concatenate([g1, g1]); b2c = jnp.concatenate([b1, b1])
    sc2 = (g2 / jnp.sqrt(var3 + 1e-5))[None, :]
    sh2 = (b2c - mean3 * g2 / jnp.sqrt(var3 + 1e-5))[None, :]

    eyeh = np.eye(NHID, dtype=np.float32)
    isum = jnp.asarray(np.concatenate([eyeh, eyeh], axis=0))
    hrep = jnp.asarray(np.kron(np.eye(NHEAD, dtype=np.float32),
                               np.ones((DH, DH), np.float32) / np.sqrt(DH)))
    tws = []
    for l in range(NL_RHO):
        p = params['tr'][l]
        tws += [p['Wq'], p['Wk'], p['Wv'], p['Wo'],
                p['ln1_g'][None, :], p['ln1_b'][None, :],
                p['ff1'], p['ff1_b'][None, :], p['ff2'], p['ff2_b'][None, :],
                p['ln2_g'][None, :], p['ln2_b'][None, :]]
    y, st4 = _t4_stage(hcat, x1cat, maskT3, sc2, sh2, isum, hrep, tws,
                       params['out_W'])
    return _t5_stage(y, st4, params['out_bn_g'], params['out_bn_b'])


# speedup vs baseline: 1.4169x; 1.4169x over previous
import functools

import jax
import jax.numpy as jnp
import numpy as np
from jax import lax
from jax.experimental import pallas as pl
from jax.experimental.pallas import tpu as pltpu
from jax.experimental.pallas import tpu_sc as plsc

N = 10000
K = 8
E = 160000
G = 64
NHID = 64
NL_RHO = 2
NHEAD = 4
DFF = 128
DH = NHID // NHEAD

NC = 2
NS = 16

_SC_MESH = plsc.VectorSubcoreMesh(core_axis_name="c", subcore_axis_name="s")

_CH = 624
_CH_TAIL = N - NS * _CH

_B = 128
_EPW0 = (E // (NC * NS * _B)) * _B
_NB0 = _EPW0 // _B
_TAIL0 = (E - NC * NS * _EPW0) // _B


def _rowchunk_copy(src_ref, dst_ref, sub):
    pltpu.sync_copy(src_ref.at[pl.ds(sub * _CH, _CH)],
                    dst_ref.at[pl.ds(sub * _CH, _CH)])

    @pl.when(sub == 0)
    def _():
        pltpu.sync_copy(src_ref.at[pl.ds(NS * _CH, _CH_TAIL)],
                        dst_ref.at[pl.ds(NS * _CH, _CH_TAIL)])


def _edge_batch(src_ref, dst_ref, table_ref, sidx, didx, gbuf, acc, gsem, base):
    pltpu.sync_copy(src_ref.at[pl.ds(base, _B)], sidx)
    pltpu.sync_copy(dst_ref.at[pl.ds(base, _B)], didx)
    pltpu.async_copy(table_ref.at[sidx], gbuf, gsem).wait()
    pltpu.sync_copy(gbuf, acc.at[didx], add=True)


def _agg0_body(ev_hbm, src_hbm, dst_hbm, zero_hbm, out_hbm,
               sidx, didx, gbuf, acc, gsem):
    core = lax.axis_index("c")
    sub = lax.axis_index("s")
    wid = sub * NC + core
    _rowchunk_copy(zero_hbm, acc, sub)
    plsc.subcore_barrier()

    def body(i, _):
        _edge_batch(src_hbm, dst_hbm, ev_hbm, sidx, didx, gbuf, acc, gsem,
                    wid * _EPW0 + i * _B)
        return 0

    lax.fori_loop(0, _NB0, body, 0)

    @pl.when(wid < _TAIL0)
    def _():
        _edge_batch(src_hbm, dst_hbm, ev_hbm, sidx, didx, gbuf, acc, gsem,
                    NC * NS * _EPW0 + wid * _B)

    plsc.subcore_barrier()
    _rowchunk_copy(acc, out_hbm.at[core], sub)


def _sc_agg0(ev, src, dst, zeros_nk):
    kfn = pl.kernel(
        _agg0_body,
        mesh=_SC_MESH,
        compiler_params=pltpu.CompilerParams(use_tc_tiling_on_sc=False),
        out_type=jax.ShapeDtypeStruct((NC, N, K), jnp.float32),
        scratch_types=[
            pltpu.VMEM((_B,), jnp.int32),
            pltpu.VMEM((_B,), jnp.int32),
            pltpu.VMEM((_B, K), jnp.float32),
            pltpu.VMEM_SHARED((N, K), jnp.float32),
            pltpu.SemaphoreType.DMA,
        ],
    )
    return kfn(ev, src, dst, zeros_nk)


def _agg1_body(x_hbm, srck_hbm, dst_hbm, zero_hbm, out_hbm,
               sidx0, didx0, gbuf0, sidx1, didx1, gbuf1, acc, sem0, sem1):
    core = lax.axis_index("c")
    sub = lax.axis_index("s")
    epw = (E // (NS * _B)) * _B
    nb = epw // _B
    tail = (E - NS * epw) // _B

    for kk in range(K // NC):
        k = core * (K // NC) + kk
        _rowchunk_copy(zero_hbm, acc, sub)
        plsc.subcore_barrier()
        srck = srck_hbm.at[k, 0]

        @pl.when(sub < tail)
        def _():
            _edge_batch(srck, dst_hbm, x_hbm, sidx0, didx0, gbuf0, acc, sem0,
                        NS * epw + sub * _B)

        def start(i, s, d, g, sem):
            base = sub * epw + i * _B
            pltpu.sync_copy(srck.at[pl.ds(base, _B)], s)
            pltpu.sync_copy(dst_hbm.at[pl.ds(base, _B)], d)
            return pltpu.async_copy(x_hbm.at[s], g, sem)

        start(0, sidx0, didx0, gbuf0, sem0)

        def body(i2, _):
            a = 2 * i2
            start(a + 1, sidx1, didx1, gbuf1, sem1)
            pltpu.make_async_copy(x_hbm.at[sidx0], gbuf0, sem0).wait()
            pltpu.sync_copy(gbuf0, acc.at[didx0], add=True)

            @pl.when(i2 < nb // 2 - 1)
            def _():
                start(a + 2, sidx0, didx0, gbuf0, sem0)

            pltpu.make_async_copy(x_hbm.at[sidx1], gbuf1, sem1).wait()
            pltpu.sync_copy(gbuf1, acc.at[didx1], add=True)
            return 0

        lax.fori_loop(0, nb // 2, body, 0)
        plsc.subcore_barrier()
        _rowchunk_copy(acc, out_hbm.at[k], sub)
        plsc.subcore_barrier()


def _sc_agg1(x1cat_flat, srck, dst, zeros_big):
    srck = srck.reshape(K, 1, E)
    kfn = pl.kernel(
        _agg1_body,
        mesh=_SC_MESH,
        compiler_params=pltpu.CompilerParams(use_tc_tiling_on_sc=False),
        out_type=jax.ShapeDtypeStruct((K, N, 2 * NHID), jnp.float32),
        scratch_types=[
            pltpu.VMEM((_B,), jnp.int32),
            pltpu.VMEM((_B,), jnp.int32),
            pltpu.VMEM((_B, 2 * NHID), jnp.float32),
            pltpu.VMEM((_B,), jnp.int32),
            pltpu.VMEM((_B,), jnp.int32),
            pltpu.VMEM((_B, 2 * NHID), jnp.float32),
            pltpu.VMEM_SHARED((N, 2 * NHID), jnp.float32),
            pltpu.SemaphoreType.DMA,
            pltpu.SemaphoreType.DMA,
        ],
    )
    return kfn(x1cat_flat, srck, dst, zeros_big)


def _dot(a, b):
    return jnp.dot(a, b, precision=jax.lax.Precision.DEFAULT)


def _dot_hi(a, b):
    return jnp.dot(a, b, precision=jax.lax.Precision.HIGHEST)


def _b16(x):
    return x.astype(jnp.bfloat16).astype(jnp.float32)

_BN1 = 2000
_BN = 400


def _t1_kernel(ev_ref, parts_ref, maskf_ref, s_ref, stats_ref):
    i = pl.program_id(0)
    s = ev_ref[...] + parts_ref[0] + parts_ref[1]
    mk = maskf_ref[...]
    sm = s * mk

    @pl.when(i == 0)
    def _():
        stats_ref[...] = jnp.zeros_like(stats_ref)

    stats_ref[0, :] += jnp.sum(sm, axis=0)
    stats_ref[1, :] += jnp.sum(sm * s, axis=0)
    stats_ref[2, :] += jnp.sum(mk, axis=0)
    s_ref[...] = s


def _t1_stage(ev, parts, maskf):
    grid = N // _BN1
    return pl.pallas_call(
        _t1_kernel,
        grid=(grid,),
        in_specs=[
            pl.BlockSpec((_BN1, K), lambda i: (i, 0)),
            pl.BlockSpec((NC, _BN1, K), lambda i: (0, i, 0)),
            pl.BlockSpec((_BN1, K), lambda i: (i, 0)),
        ],
        out_specs=[
            pl.BlockSpec((_BN1, K), lambda i: (i, 0)),
            pl.BlockSpec((3, K), lambda i: (0, 0)),
        ],
        out_shape=[
            jax.ShapeDtypeStruct((N, K), jnp.float32),
            jax.ShapeDtypeStruct((3, K), jnp.float32),
        ],
    )(ev, parts, maskf)


def _t2_kernel(sT_ref, maskT_ref, a2_ref, b2_ref, x1_ref):
    s3 = sT_ref[...]
    y = s3 * a2_ref[0, :] + b2_ref[0, :]
    x1_ref[...] = jnp.maximum(y, 0.0) * maskT_ref[...]


def _t2_stage(sT, maskT, a2, b2):
    grid = N // _BN
    return pl.pallas_call(
        _t2_kernel,
        grid=(grid,),
        in_specs=[
            pl.BlockSpec((K, _BN, 1), lambda i: (0, i, 0)),
            pl.BlockSpec((K, _BN, 1), lambda i: (0, i, 0)),
            pl.BlockSpec((1, 2 * NHID), lambda i: (0, 0)),
            pl.BlockSpec((1, 2 * NHID), lambda i: (0, 0)),
        ],
        out_specs=pl.BlockSpec((K, _BN, 2 * NHID), lambda i: (0, i, 0)),
        out_shape=jax.ShapeDtypeStruct((K, N, 2 * NHID), jnp.float32),
    )(sT, maskT, a2, b2)


def _t3_kernel(x1_ref, agg_ref, w2_ref, maskT_ref, hcat_ref, stats_ref):
    i = pl.program_id(0)
    xs = x1_ref[...] + agg_ref[...]
    h = _dot(xs.reshape(K * _BN, 2 * NHID), w2_ref[...]).reshape(K, _BN, 2 * NHID)
    h = h * maskT_ref[...]

    @pl.when(i == 0)
    def _():
        stats_ref[...] = jnp.zeros_like(stats_ref)

    hf = h.reshape(K * _BN, 2 * NHID)
    stats_ref[0, :] += jnp.sum(hf, axis=0)
    stats_ref[1, :] += jnp.sum(hf * hf, axis=0)
    hcat_ref[...] = h


def _t3_stage(x1cat, agg1, w2, maskT):
    grid = N // _BN
    return pl.pallas_call(
        _t3_kernel,
        grid=(grid,),
        in_specs=[
            pl.BlockSpec((K, _BN, 2 * NHID), lambda i: (0, i, 0)),
            pl.BlockSpec((K, _BN, 2 * NHID), lambda i: (0, i, 0)),
            pl.BlockSpec((2 * NHID, 2 * NHID), lambda i: (0, 0)),
            pl.BlockSpec((K, _BN, 1), lambda i: (0, i, 0)),
        ],
        out_specs=[
            pl.BlockSpec((K, _BN, 2 * NHID), lambda i: (0, i, 0)),
            pl.BlockSpec((2, 2 * NHID), lambda i: (0, 0)),
        ],
        out_shape=[
            jax.ShapeDtypeStruct((K, N, 2 * NHID), jnp.float32),
            jax.ShapeDtypeStruct((2, 2 * NHID), jnp.float32),
        ],
    )(x1cat, agg1, w2, maskT)


def _layernorm_k(x, g, b):
    m = jnp.mean(x, axis=-1, keepdims=True)
    d = x - m
    v = jnp.mean(d * d, axis=-1, keepdims=True)
    return d / jnp.sqrt(v + 1e-5) * g + b


def _t4_kernel(hcat_ref, x1_ref, maskT_ref, sc2_ref, sh2_ref,
               isum_ref, hrep_ref, tw_refs, outw_ref, y_ref, stats_ref):
    i = pl.program_id(0)
    mk3 = maskT_ref[...]
    x2 = jnp.maximum(hcat_ref[...] * sc2_ref[0, :] + sh2_ref[0, :], 0.0)
    x2 = x2 * mk3 + x1_ref[...]
    x = x2[..., :NHID] + x2[..., NHID:]

    for l in range(NL_RHO):
        (wq, wk, wv, wo, ln1g, ln1b, ff1, ff1b, ff2, ff2b,
         ln2g, ln2b) = tw_refs[12 * l:12 * (l + 1)]
        xf = x.reshape(K * _BN, NHID)
        q = _dot(xf, wq[...]).reshape(K, _BN, NHID)
        kk = _dot(xf, wk[...]).reshape(K, _BN, NHID)
        v = _dot(xf, wv[...]).reshape(K, _BN, NHID)
        qb = _b16(q)
        kb = _b16(kk)
        sc = []
        for j in range(K):
            prod = (qb * kb[j][None, :, :]).reshape(K * _BN, NHID)
            ph = _b16(prod)
            plo = prod - ph
            sj = (_dot(ph, hrep_ref[...])
                  + _dot(plo, hrep_ref[...])).reshape(K, _BN, NHID)
            mj = mk3[j][None, :, :]
            sc.append(jnp.where(mj > 0.0, sj, -1e9))
        m8 = sc[0]
        for j in range(1, K):
            m8 = jnp.maximum(m8, sc[j])
        ssum = None
        es = []
        for j in range(K):
            ej = jnp.exp(sc[j] - m8)
            es.append(ej)
            ssum = ej if ssum is None else ssum + ej
        vb = _b16(v)
        o = None
        for j in range(K):
            contrib = _b16(es[j] / ssum) * vb[j][None, :, :]
            o = contrib if o is None else o + contrib
        o = _dot(o.reshape(K * _BN, NHID), wo[...]).reshape(K, _BN, NHID)
        x = _layernorm_k(x + o, ln1g[0, :], ln1b[0, :])
        xf = x.reshape(K * _BN, NHID)
        f = jnp.maximum(_dot(xf, ff1[...]) + ff1b[0, :], 0.0)
        f = (_dot(f, ff2[...]) + ff2b[0, :]).reshape(K, _BN, NHID)
        x = _layernorm_k(x + f, ln2g[0, :], ln2b[0, :])
        x = x * mk3

    hsum = x[0]
    for j in range(1, K):
        hsum = hsum + x[j]
    y = _dot(hsum, outw_ref[...])

    @pl.when(i == 0)
    def _():
        stats_ref[...] = jnp.zeros_like(stats_ref)

    stats_ref[0, :] += jnp.sum(y, axis=0)
    stats_ref[1, :] += jnp.sum(y * y, axis=0)
    y_ref[...] = y


def _t4_stage(hcat, x1cat, maskT, sc2, sh2, isum, hrep, tws, outw):
    grid = N // _BN
    full = lambda shape: pl.BlockSpec(shape, lambda i: tuple(0 for _ in shape))
    nblk3 = pl.BlockSpec((K, _BN, 2 * NHID), lambda i: (0, i, 0))

    def body(hcat_ref, x1_ref, maskT_ref, sc2_ref, sh2_ref, isum_ref,
             hrep_ref, *rest):
        tw_refs = rest[:-3]
        outw_ref, y_ref, stats_ref = rest[-3:]
        _t4_kernel(hcat_ref, x1_ref, maskT_ref, sc2_ref, sh2_ref,
                   isum_ref, hrep_ref, tw_refs, outw_ref, y_ref, stats_ref)

    in_specs = [
        nblk3, nblk3,
        pl.BlockSpec((K, _BN, 1), lambda i: (0, i, 0)),
        full((1, 2 * NHID)), full((1, 2 * NHID)),
        full((2 * NHID, NHID)), full((NHID, NHID)),
    ] + [full(t.shape) for t in tws] + [full((NHID, NHID))]
    return pl.pallas_call(
        body,
        grid=(grid,),
        in_specs=in_specs,
        out_specs=[
            pl.BlockSpec((_BN, NHID), lambda i: (i, 0)),
            pl.BlockSpec((2, NHID), lambda i: (0, 0)),
        ],
        out_shape=[
            jax.ShapeDtypeStruct((N, NHID), jnp.float32),
            jax.ShapeDtypeStruct((2, NHID), jnp.float32),
        ],
    )(hcat, x1cat, maskT, sc2, sh2, isum, hrep, *tws, outw)


def _t5_kernel(y_ref, stat_ref, g_ref, b_ref, o_ref):
    s1 = stat_ref[0, :]
    s2 = stat_ref[1, :]
    mean = s1 / N
    var = s2 / N - mean * mean
    o_ref[...] = (y_ref[...] - mean) / jnp.sqrt(var + 1e-5) * g_ref[...] + b_ref[...]


def _t5_stage(y, stats, g, b):
    grid = N // _BN1
    return pl.pallas_call(
        _t5_kernel,
        grid=(grid,),
        in_specs=[
            pl.BlockSpec((_BN1, NHID), lambda i: (i, 0)),
            pl.BlockSpec((2, NHID), lambda i: (0, 0)),
            pl.BlockSpec((NHID,), lambda i: (0,)),
            pl.BlockSpec((NHID,), lambda i: (0,)),
        ],
        out_specs=pl.BlockSpec((_BN1, NHID), lambda i: (i, 0)),
        out_shape=jax.ShapeDtypeStruct((N, NHID), jnp.float32),
    )(y, stats, g, b)


def kernel(eigen_vectors, eigen_values, batch, edge_index, params):
    f32 = jnp.float32
    ev = eigen_vectors.astype(f32)
    src = edge_index[0].astype(jnp.int32)
    dst = edge_index[1].astype(jnp.int32)
    batch = batch.astype(jnp.int32)

    size = (jnp.searchsorted(batch, jnp.arange(1, G + 1, dtype=jnp.int32))
            - jnp.searchsorted(batch, jnp.arange(G, dtype=jnp.int32)))
    cntn = size[batch]
    ar = jnp.arange(K, dtype=jnp.int32)
    maskf = (ar[None, :] < cntn[:, None]).astype(f32)
    maskT3 = (ar[:, None] < cntn[None, :]).astype(f32)[..., None]

    zeros_nk = jnp.zeros((N, K), f32)
    parts = _sc_agg0(ev, src, dst, zeros_nk)
    s, st1 = _t1_stage(ev, parts, maskf)
    s1 = jnp.sum(st1[0]); s2 = jnp.sum(st1[1])
    cnt = jnp.maximum(jnp.sum(st1[2]), 1.0)

    w0 = params['conv_W'][0][0]
    g0 = params['bn_g'][0]; b0 = params['bn_b'][0]
    mu = s1 / cnt
    var0 = w0 * w0 * jnp.maximum(s2 / cnt - mu * mu, 0.0)
    inv0 = g0 / jnp.sqrt(var0 + 1e-5)
    a_p = w0 * inv0
    b_p = b0 - w0 * mu * inv0
    b_m = b0 + w0 * mu * inv0
    a2 = jnp.concatenate([a_p, -a_p])[None, :]
    b2 = jnp.concatenate([b_p, b_m])[None, :]

    x1cat = _t2_stage(s.T[..., None], maskT3, a2, b2)

    srck = src[None, :] + (jnp.arange(K, dtype=jnp.int32) * N)[:, None]
    zeros_big = jnp.zeros((N, 2 * NHID), f32)
    agg1 = _sc_agg1(x1cat.reshape(K * N, 2 * NHID), srck, dst, zeros_big)

    w1 = params['conv_W'][1]
    zz = jnp.zeros_like(w1)
    w2 = jnp.block([[w1, zz], [zz, w1]])
    hcat, st3 = _t3_stage(x1cat, agg1, w2, maskT3)

    g1 = params['bn_g'][1]; b1 = params['bn_b'][1]
    mean3 = st3[0] / cnt
    var3 = jnp.maximum(st3[1] / cnt - mean3 * mean3, 0.0)
    g2 = jnp.concatenate([g1, g1]); b2c = jnp.concatenate([b1, b1])
    sc2 = (g2 / jnp.sqrt(var3 + 1e-5))[None, :]
    sh2 = (b2c - mean3 * g2 / jnp.sqrt(var3 + 1e-5))[None, :]

    eyeh = np.eye(NHID, dtype=np.float32)
    isum = jnp.asarray(np.concatenate([eyeh, eyeh], axis=0))
    hrep = jnp.asarray(np.kron(np.eye(NHEAD, dtype=np.float32),
                               np.ones((DH, DH), np.float32) / np.sqrt(DH)))
    tws = []
    for l in range(NL_RHO):
        p = params['tr'][l]
        tws += [p['Wq'], p['Wk'], p['Wv'], p['Wo'],
                p['ln1_g'][None, :], p['ln1_b'][None, :],
                p['ff1'], p['ff1_b'][None, :], p['ff2'], p['ff2_b'][None, :],
                p['ln2_g'][None, :], p['ln2_b'][None, :]]
    y, st4 = _t4_stage(hcat, x1cat, maskT3, sc2, sh2, isum, hrep, tws,
                       params['out_W'])
    return _t5_stage(y, st4, params['out_bn_g'], params['out_bn_b'])


# speedup vs baseline: 1.5637x; 1.1036x over previous
import functools

import jax
import jax.numpy as jnp
import numpy as np
from jax import lax
from jax.experimental import pallas as pl
from jax.experimental.pallas import tpu as pltpu
from jax.experimental.pallas import tpu_sc as plsc

N = 10000
K = 8
E = 160000
G = 64
NHID = 64
NL_RHO = 2
NHEAD = 4
DFF = 128
DH = NHID // NHEAD

NC = 2
NS = 16

_SC_MESH = plsc.VectorSubcoreMesh(core_axis_name="c", subcore_axis_name="s")

_CH = 624
_CH_TAIL = N - NS * _CH

_B = 128
_EPW0 = (E // (NC * NS * _B)) * _B
_NB0 = _EPW0 // _B
_TAIL0 = (E - NC * NS * _EPW0) // _B


def _rowchunk_copy(src_ref, dst_ref, sub):
    pltpu.sync_copy(src_ref.at[pl.ds(sub * _CH, _CH)],
                    dst_ref.at[pl.ds(sub * _CH, _CH)])

    @pl.when(sub == 0)
    def _():
        pltpu.sync_copy(src_ref.at[pl.ds(NS * _CH, _CH_TAIL)],
                        dst_ref.at[pl.ds(NS * _CH, _CH_TAIL)])


def _edge_batch(src_ref, dst_ref, table_ref, sidx, didx, gbuf, acc, gsem, base):
    pltpu.sync_copy(src_ref.at[pl.ds(base, _B)], sidx)
    pltpu.sync_copy(dst_ref.at[pl.ds(base, _B)], didx)
    pltpu.async_copy(table_ref.at[sidx], gbuf, gsem).wait()
    pltpu.sync_copy(gbuf, acc.at[didx], add=True)


def _agg0_body(ev_hbm, src_hbm, dst_hbm, zero_hbm, out_hbm,
               sidx, didx, gbuf, acc, gsem):
    core = lax.axis_index("c")
    sub = lax.axis_index("s")
    wid = sub * NC + core
    _rowchunk_copy(zero_hbm, acc, sub)
    plsc.subcore_barrier()

    def body(i, _):
        _edge_batch(src_hbm, dst_hbm, ev_hbm, sidx, didx, gbuf, acc, gsem,
                    wid * _EPW0 + i * _B)
        return 0

    lax.fori_loop(0, _NB0, body, 0)

    @pl.when(wid < _TAIL0)
    def _():
        _edge_batch(src_hbm, dst_hbm, ev_hbm, sidx, didx, gbuf, acc, gsem,
                    NC * NS * _EPW0 + wid * _B)

    plsc.subcore_barrier()
    _rowchunk_copy(acc, out_hbm.at[core], sub)


def _sc_agg0(ev, src, dst, zeros_nk):
    kfn = pl.kernel(
        _agg0_body,
        mesh=_SC_MESH,
        compiler_params=pltpu.CompilerParams(use_tc_tiling_on_sc=False),
        out_type=jax.ShapeDtypeStruct((NC, N, K), jnp.float32),
        scratch_types=[
            pltpu.VMEM((_B,), jnp.int32),
            pltpu.VMEM((_B,), jnp.int32),
            pltpu.VMEM((_B, K), jnp.float32),
            pltpu.VMEM_SHARED((N, K), jnp.float32),
            pltpu.SemaphoreType.DMA,
        ],
    )
    return kfn(ev, src, dst, zeros_nk)


def _agg1_body(x_hbm, srck_hbm, dst_hbm, zero_hbm, out_hbm,
               sidx_all, didx0, didx1, gbuf0, gbuf1, acc, sem0, sem1):
    core = lax.axis_index("c")
    sub = lax.axis_index("s")
    epw = (E // (NS * _B)) * _B
    nb = epw // _B
    tail = (E - NS * epw) // _B

    for kk in range(K // NC):
        k = core * (K // NC) + kk
        _rowchunk_copy(zero_hbm, acc, sub)
        pltpu.sync_copy(srck_hbm.at[k, pl.ds(sub * nb, nb)], sidx_all)
        plsc.subcore_barrier()
        srck1 = srck_hbm.at[k]

        @pl.when(sub < tail)
        def _():
            trow = NS * nb + sub
            pltpu.sync_copy(dst_hbm.at[trow, 0], didx0)
            pltpu.sync_copy(srck1.at[trow], didx1)
            pltpu.async_copy(x_hbm.at[didx1], gbuf0, sem0).wait()
            pltpu.sync_copy(gbuf0, acc.at[didx0], add=True)

        def start(i, g, sem):
            return pltpu.async_copy(x_hbm.at[sidx_all.at[i]], g, sem)

        start(0, gbuf0, sem0)

        def body(i2, _):
            a = 2 * i2
            start(a + 1, gbuf1, sem1)
            pltpu.sync_copy(dst_hbm.at[sub * nb + a, 0], didx0)
            pltpu.make_async_copy(x_hbm.at[sidx_all.at[a]], gbuf0, sem0).wait()
            pltpu.sync_copy(gbuf0, acc.at[didx0], add=True)

            @pl.when(i2 < nb // 2 - 1)
            def _():
                start(a + 2, gbuf0, sem0)

            pltpu.sync_copy(dst_hbm.at[sub * nb + a + 1, 0], didx1)
            pltpu.make_async_copy(x_hbm.at[sidx_all.at[a + 1]], gbuf1,
                                  sem1).wait()
            pltpu.sync_copy(gbuf1, acc.at[didx1], add=True)
            return 0

        lax.fori_loop(0, nb // 2, body, 0)
        plsc.subcore_barrier()
        _rowchunk_copy(acc, out_hbm.at[k], sub)
        plsc.subcore_barrier()


def _sc_agg1(x1cat_flat, srck, dst, zeros_big):
    nrows = E // _B
    srck = srck.reshape(K, nrows, _B)
    dst = dst.reshape(nrows, 1, _B)
    nb = (E // (NS * _B))
    kfn = pl.kernel(
        _agg1_body,
        mesh=_SC_MESH,
        compiler_params=pltpu.CompilerParams(use_tc_tiling_on_sc=False),
        out_type=jax.ShapeDtypeStruct((K, N, 2 * NHID), jnp.float32),
        scratch_types=[
            pltpu.VMEM((nb, _B), jnp.int32),
            pltpu.VMEM((_B,), jnp.int32),
            pltpu.VMEM((_B,), jnp.int32),
            pltpu.VMEM((_B, 2 * NHID), jnp.float32),
            pltpu.VMEM((_B, 2 * NHID), jnp.float32),
            pltpu.VMEM_SHARED((N, 2 * NHID), jnp.float32),
            pltpu.SemaphoreType.DMA,
            pltpu.SemaphoreType.DMA,
        ],
    )
    return kfn(x1cat_flat, srck, dst, zeros_big)


def _dot(a, b):
    return jnp.dot(a, b, precision=jax.lax.Precision.DEFAULT)


def _dot_hi(a, b):
    return jnp.dot(a, b, precision=jax.lax.Precision.HIGHEST)


def _b16(x):
    return x.astype(jnp.bfloat16).astype(jnp.float32)

_BN1 = 2000
_BN = 400


def _t1_kernel(ev_ref, parts_ref, maskf_ref, s_ref, stats_ref):
    i = pl.program_id(0)
    s = ev_ref[...] + parts_ref[0] + parts_ref[1]
    mk = maskf_ref[...]
    sm = s * mk

    @pl.when(i == 0)
    def _():
        stats_ref[...] = jnp.zeros_like(stats_ref)

    stats_ref[0, :] += jnp.sum(sm, axis=0)
    stats_ref[1, :] += jnp.sum(sm * s, axis=0)
    stats_ref[2, :] += jnp.sum(mk, axis=0)
    s_ref[...] = s


def _t1_stage(ev, parts, maskf):
    grid = N // _BN1
    return pl.pallas_call(
        _t1_kernel,
        grid=(grid,),
        in_specs=[
            pl.BlockSpec((_BN1, K), lambda i: (i, 0)),
            pl.BlockSpec((NC, _BN1, K), lambda i: (0, i, 0)),
            pl.BlockSpec((_BN1, K), lambda i: (i, 0)),
        ],
        out_specs=[
            pl.BlockSpec((_BN1, K), lambda i: (i, 0)),
            pl.BlockSpec((3, K), lambda i: (0, 0)),
        ],
        out_shape=[
            jax.ShapeDtypeStruct((N, K), jnp.float32),
            jax.ShapeDtypeStruct((3, K), jnp.float32),
        ],
    )(ev, parts, maskf)


def _t2_kernel(sT_ref, maskT_ref, a2_ref, b2_ref, x1_ref):
    s3 = sT_ref[...]
    y = s3 * a2_ref[0, :] + b2_ref[0, :]
    x1_ref[...] = jnp.maximum(y, 0.0) * maskT_ref[...]


def _t2_stage(sT, maskT, a2, b2):
    grid = N // _BN
    return pl.pallas_call(
        _t2_kernel,
        grid=(grid,),
        in_specs=[
            pl.BlockSpec((K, _BN, 1), lambda i: (0, i, 0)),
            pl.BlockSpec((K, _BN, 1), lambda i: (0, i, 0)),
            pl.BlockSpec((1, 2 * NHID), lambda i: (0, 0)),
            pl.BlockSpec((1, 2 * NHID), lambda i: (0, 0)),
        ],
        out_specs=pl.BlockSpec((K, _BN, 2 * NHID), lambda i: (0, i, 0)),
        out_shape=jax.ShapeDtypeStruct((K, N, 2 * NHID), jnp.float32),
    )(sT, maskT, a2, b2)


def _t3_kernel(x1_ref, agg_ref, w2_ref, maskT_ref, hcat_ref, stats_ref):
    i = pl.program_id(0)
    xs = x1_ref[...] + agg_ref[...]
    h = _dot(xs.reshape(K * _BN, 2 * NHID), w2_ref[...]).reshape(K, _BN, 2 * NHID)
    h = h * maskT_ref[...]

    @pl.when(i == 0)
    def _():
        stats_ref[...] = jnp.zeros_like(stats_ref)

    hf = h.reshape(K * _BN, 2 * NHID)
    stats_ref[0, :] += jnp.sum(hf, axis=0)
    stats_ref[1, :] += jnp.sum(hf * hf, axis=0)
    hcat_ref[...] = h


def _t3_stage(x1cat, agg1, w2, maskT):
    grid = N // _BN
    return pl.pallas_call(
        _t3_kernel,
        grid=(grid,),
        in_specs=[
            pl.BlockSpec((K, _BN, 2 * NHID), lambda i: (0, i, 0)),
            pl.BlockSpec((K, _BN, 2 * NHID), lambda i: (0, i, 0)),
            pl.BlockSpec((2 * NHID, 2 * NHID), lambda i: (0, 0)),
            pl.BlockSpec((K, _BN, 1), lambda i: (0, i, 0)),
        ],
        out_specs=[
            pl.BlockSpec((K, _BN, 2 * NHID), lambda i: (0, i, 0)),
            pl.BlockSpec((2, 2 * NHID), lambda i: (0, 0)),
        ],
        out_shape=[
            jax.ShapeDtypeStruct((K, N, 2 * NHID), jnp.float32),
            jax.ShapeDtypeStruct((2, 2 * NHID), jnp.float32),
        ],
    )(x1cat, agg1, w2, maskT)


def _layernorm_k(x, g, b):
    m = jnp.mean(x, axis=-1, keepdims=True)
    d = x - m
    v = jnp.mean(d * d, axis=-1, keepdims=True)
    return d / jnp.sqrt(v + 1e-5) * g + b


def _t4_kernel(hcat_ref, x1_ref, maskT_ref, sc2_ref, sh2_ref,
               isum_ref, hrep_ref, tw_refs, outw_ref, y_ref, stats_ref):
    i = pl.program_id(0)
    mk3 = maskT_ref[...]
    x2 = jnp.maximum(hcat_ref[...] * sc2_ref[0, :] + sh2_ref[0, :], 0.0)
    x2 = x2 * mk3 + x1_ref[...]
    x = x2[..., :NHID] + x2[..., NHID:]

    for l in range(NL_RHO):
        (wq, wk, wv, wo, ln1g, ln1b, ff1, ff1b, ff2, ff2b,
         ln2g, ln2b) = tw_refs[12 * l:12 * (l + 1)]
        xf = x.reshape(K * _BN, NHID)
        q = _dot(xf, wq[...]).reshape(K, _BN, NHID)
        kk = _dot(xf, wk[...]).reshape(K, _BN, NHID)
        v = _dot(xf, wv[...]).reshape(K, _BN, NHID)
        qb = _b16(q)
        kb = _b16(kk)
        sc = []
        for j in range(K):
            prod = (qb * kb[j][None, :, :]).reshape(K * _BN, NHID)
            ph = _b16(prod)
            plo = prod - ph
            sj = (_dot(ph, hrep_ref[...])
                  + _dot(plo, hrep_ref[...])).reshape(K, _BN, NHID)
            mj = mk3[j][None, :, :]
            sc.append(jnp.where(mj > 0.0, sj, -1e9))
        m8 = sc[0]
        for j in range(1, K):
            m8 = jnp.maximum(m8, sc[j])
        ssum = None
        es = []
        for j in range(K):
            ej = jnp.exp(sc[j] - m8)
            es.append(ej)
            ssum = ej if ssum is None else ssum + ej
        vb = _b16(v)
        o = None
        for j in range(K):
            contrib = _b16(es[j] / ssum) * vb[j][None, :, :]
            o = contrib if o is None else o + contrib
        o = _dot(o.reshape(K * _BN, NHID), wo[...]).reshape(K, _BN, NHID)
        x = _layernorm_k(x + o, ln1g[0, :], ln1b[0, :])
        xf = x.reshape(K * _BN, NHID)
        f = jnp.maximum(_dot(xf, ff1[...]) + ff1b[0, :], 0.0)
        f = (_dot(f, ff2[...]) + ff2b[0, :]).reshape(K, _BN, NHID)
        x = _layernorm_k(x + f, ln2g[0, :], ln2b[0, :])
        x = x * mk3

    hsum = x[0]
    for j in range(1, K):
        hsum = hsum + x[j]
    y = _dot(hsum, outw_ref[...])

    @pl.when(i == 0)
    def _():
        stats_ref[...] = jnp.zeros_like(stats_ref)

    stats_ref[0, :] += jnp.sum(y, axis=0)
    stats_ref[1, :] += jnp.sum(y * y, axis=0)
    y_ref[...] = y


def _t4_stage(hcat, x1cat, maskT, sc2, sh2, isum, hrep, tws, outw):
    grid = N // _BN
    full = lambda shape: pl.BlockSpec(shape, lambda i: tuple(0 for _ in shape))
    nblk3 = pl.BlockSpec((K, _BN, 2 * NHID), lambda i: (0, i, 0))

    def body(hcat_ref, x1_ref, maskT_ref, sc2_ref, sh2_ref, isum_ref,
             hrep_ref, *rest):
        tw_refs = rest[:-3]
        outw_ref, y_ref, stats_ref = rest[-3:]
        _t4_kernel(hcat_ref, x1_ref, maskT_ref, sc2_ref, sh2_ref,
                   isum_ref, hrep_ref, tw_refs, outw_ref, y_ref, stats_ref)

    in_specs = [
        nblk3, nblk3,
        pl.BlockSpec((K, _BN, 1), lambda i: (0, i, 0)),
        full((1, 2 * NHID)), full((1, 2 * NHID)),
        full((2 * NHID, NHID)), full((NHID, NHID)),
    ] + [full(t.shape) for t in tws] + [full((NHID, NHID))]
    return pl.pallas_call(
        body,
        grid=(grid,),
        in_specs=in_specs,
        out_specs=[
            pl.BlockSpec((_BN, NHID), lambda i: (i, 0)),
            pl.BlockSpec((2, NHID), lambda i: (0, 0)),
        ],
        out_shape=[
            jax.ShapeDtypeStruct((N, NHID), jnp.float32),
            jax.ShapeDtypeStruct((2, NHID), jnp.float32),
        ],
    )(hcat, x1cat, maskT, sc2, sh2, isum, hrep, *tws, outw)


def _t5_kernel(y_ref, stat_ref, g_ref, b_ref, o_ref):
    s1 = stat_ref[0, :]
    s2 = stat_ref[1, :]
    mean = s1 / N
    var = s2 / N - mean * mean
    o_ref[...] = (y_ref[...] - mean) / jnp.sqrt(var + 1e-5) * g_ref[...] + b_ref[...]


def _t5_stage(y, stats, g, b):
    grid = N // _BN1
    return pl.pallas_call(
        _t5_kernel,
        grid=(grid,),
        in_specs=[
            pl.BlockSpec((_BN1, NHID), lambda i: (i, 0)),
            pl.BlockSpec((2, NHID), lambda i: (0, 0)),
            pl.BlockSpec((NHID,), lambda i: (0,)),
            pl.BlockSpec((NHID,), lambda i: (0,)),
        ],
        out_specs=pl.BlockSpec((_BN1, NHID), lambda i: (i, 0)),
        out_shape=jax.ShapeDtypeStruct((N, NHID), jnp.float32),
    )(y, stats, g, b)


def kernel(eigen_vectors, eigen_values, batch, edge_index, params):
    f32 = jnp.float32
    ev = eigen_vectors.astype(f32)
    src = edge_index[0].astype(jnp.int32)
    dst = edge_index[1].astype(jnp.int32)
    batch = batch.astype(jnp.int32)

    size = (jnp.searchsorted(batch, jnp.arange(1, G + 1, dtype=jnp.int32))
            - jnp.searchsorted(batch, jnp.arange(G, dtype=jnp.int32)))
    cntn = size[batch]
    ar = jnp.arange(K, dtype=jnp.int32)
    maskf = (ar[None, :] < cntn[:, None]).astype(f32)
    maskT3 = (ar[:, None] < cntn[None, :]).astype(f32)[..., None]

    zeros_nk = jnp.zeros((N, K), f32)
    parts = _sc_agg0(ev, src, dst, zeros_nk)
    s, st1 = _t1_stage(ev, parts, maskf)
    s1 = jnp.sum(st1[0]); s2 = jnp.sum(st1[1])
    cnt = jnp.maximum(jnp.sum(st1[2]), 1.0)

    w0 = params['conv_W'][0][0]
    g0 = params['bn_g'][0]; b0 = params['bn_b'][0]
    mu = s1 / cnt
    var0 = w0 * w0 * jnp.maximum(s2 / cnt - mu * mu, 0.0)
    inv0 = g0 / jnp.sqrt(var0 + 1e-5)
    a_p = w0 * inv0
    b_p = b0 - w0 * mu * inv0
    b_m = b0 + w0 * mu * inv0
    a2 = jnp.concatenate([a_p, -a_p])[None, :]
    b2 = jnp.concatenate([b_p, b_m])[None, :]

    x1cat = _t2_stage(s.T[..., None], maskT3, a2, b2)

    srck = src[None, :] + (jnp.arange(K, dtype=jnp.int32) * N)[:, None]
    zeros_big = jnp.zeros((N, 2 * NHID), f32)
    agg1 = _sc_agg1(x1cat.reshape(K * N, 2 * NHID), srck, dst, zeros_big)

    w1 = params['conv_W'][1]
    zz = jnp.zeros_like(w1)
    w2 = jnp.block([[w1, zz], [zz, w1]])
    hcat, st3 = _t3_stage(x1cat, agg1, w2, maskT3)

    g1 = params['bn_g'][1]; b1 = params['bn_b'][1]
    mean3 = st3[0] / cnt
    var3 = jnp.maximum(st3[1] / cnt - mean3 * mean3, 0.0)
    g2 = jnp.concatenate([g1, g1]); b2c = jnp.concatenate([b1, b1])
    sc2 = (g2 / jnp.sqrt(var3 + 1e-5))[None, :]
    sh2 = (b2c - mean3 * g2 / jnp.sqrt(var3 + 1e-5))[None, :]

    eyeh = np.eye(NHID, dtype=np.float32)
    isum = jnp.asarray(np.concatenate([eyeh, eyeh], axis=0))
    hrep = jnp.asarray(np.kron(np.eye(NHEAD, dtype=np.float32),
                               np.ones((DH, DH), np.float32) / np.sqrt(DH)))
    tws = []
    for l in range(NL_RHO):
        p = params['tr'][l]
        tws += [p['Wq'], p['Wk'], p['Wv'], p['Wo'],
                p['ln1_g'][None, :], p['ln1_b'][None, :],
                p['ff1'], p['ff1_b'][None, :], p['ff2'], p['ff2_b'][None, :],
                p['ln2_g'][None, :], p['ln2_b'][None, :]]
    y, st4 = _t4_stage(hcat, x1cat, maskT3, sc2, sh2, isum, hrep, tws,
                       params['out_W'])
    return _t5_stage(y, st4, params['out_bn_g'], params['out_bn_b'])
